# baseline (speedup 1.0000x reference)
import functools

import jax
import jax.numpy as jnp
from jax import lax
from jax.experimental import pallas as pl
from jax.experimental.pallas import tpu as pltpu
from jax.experimental.pallas import tpu_sc as plsc

_LANES = 16
_N = 256
_TABLE = 2048
_LO = -3.0
_WIDTH = 6.0 / _TABLE
_SCALE = _TABLE / 6.0
_TC_ROWS = 768
_TC_BLK = 256


@functools.lru_cache(maxsize=None)
def _make_sc_quantize(batch: int):
    try:
        info = plsc.get_sparse_core_info()
        num_cores, num_subcores = info.num_cores, info.num_subcores
    except Exception:
        num_cores, num_subcores = 2, 16
    num_workers = num_cores * num_subcores
    assert batch % (num_workers * _LANES) == 0
    b_per_w = batch // num_workers
    half = b_per_w // 2
    widths = []
    w = _N // 2
    while w >= 1:
        widths.append(w)
        w //= 2

    mesh = plsc.VectorSubcoreMesh(
        core_axis_name="c",
        subcore_axis_name="s",
        num_cores=num_cores,
        num_subcores=num_subcores,
    )

    @functools.partial(
        pl.kernel,
        out_type=jax.ShapeDtypeStruct((batch,), jnp.float32),
        mesh=mesh,
        scratch_types=[
            pltpu.VMEM((b_per_w,), jnp.float32),
            pltpu.VMEM((b_per_w,), jnp.float32),
            pltpu.VMEM((_N + _LANES,), jnp.float32),
            pltpu.VMEM((_N,), jnp.float32),
            pltpu.VMEM((_TABLE,), jnp.int32),
            pltpu.VMEM((_TABLE,), jnp.float32),
            pltpu.SemaphoreType.DMA,
            pltpu.SemaphoreType.DMA,
        ],
        compiler_params=pltpu.CompilerParams(needs_layout_passes=False),
    )
    def quantize(
        x_hbm, centers_hbm, out_hbm,
        x_v, o_v, cen_v, bnd_v, start_v, bval_v, sem_in, sem_out,
    ):
        wid = lax.axis_index("s") * num_cores + lax.axis_index("c")
        base = wid * b_per_w
        in_copy = pltpu.async_copy(x_hbm.at[pl.ds(base, b_per_w)], x_v, sem_in)
        pltpu.sync_copy(centers_hbm, cen_v.at[pl.ds(0, _N)])

        lane = lax.iota(jnp.int32, _LANES)
        inf = jnp.full((_LANES,), jnp.inf, jnp.float32)

        for j in range(_N // _LANES):
            lo = cen_v[pl.ds(j * _LANES, _LANES)]
            hi = plsc.load_gather(cen_v, [lane + (j * _LANES + 1)])
            mid = (lo + hi) * 0.5
            if j == _N // _LANES - 1:
                mid = jnp.where(lane == _LANES - 1, inf, mid)
            bnd_v[pl.ds(j * _LANES, _LANES)] = mid

        @plsc.parallel_loop(0, _TABLE // _LANES, 1, unroll=4)
        def _(j):
            g = (j * _LANES + lane).astype(jnp.float32) * _WIDTH + _LO
            pos = jnp.zeros((_LANES,), jnp.int32)
            for w in widths:
                mv = plsc.load_gather(bnd_v, [pos + (w - 1)])
                pos = jnp.where(mv < g, pos + w, pos)
            start_v[pl.ds(j * _LANES, _LANES)] = pos
            bval_v[pl.ds(j * _LANES, _LANES)] = plsc.load_gather(bnd_v, [pos])

        in_copy.wait()

        @plsc.parallel_loop(0, half // _LANES, 1, unroll=8)
        def _(i):
            x = x_v[pl.ds(i * _LANES, _LANES)]
            t = jnp.clip(((x - _LO) * _SCALE).astype(jnp.int32), 0, _TABLE - 1)
            s = plsc.load_gather(start_v, [t])
            bv = plsc.load_gather(bval_v, [t])
            pos = jnp.where(bv < x, s + 1, s)
            o_v[pl.ds(i * _LANES, _LANES)] = plsc.load_gather(cen_v, [pos])

        out_copy1 = pltpu.async_copy(
            o_v.at[pl.ds(0, half)], out_hbm.at[pl.ds(base, half)], sem_out
        )

        @plsc.parallel_loop(half // _LANES, b_per_w // _LANES, 1, unroll=8)
        def _(i):
            x = x_v[pl.ds(i * _LANES, _LANES)]
            t = jnp.clip(((x - _LO) * _SCALE).astype(jnp.int32), 0, _TABLE - 1)
            s = plsc.load_gather(start_v, [t])
            bv = plsc.load_gather(bval_v, [t])
            pos = jnp.where(bv < x, s + 1, s)
            o_v[pl.ds(i * _LANES, _LANES)] = plsc.load_gather(cen_v, [pos])

        out_copy2 = pltpu.async_copy(
            o_v.at[pl.ds(half, half)], out_hbm.at[pl.ds(base + half, half)], sem_out
        )
        out_copy1.wait()
        out_copy2.wait()

    return quantize


def _tc_body(x_ref, c_ref, o_ref):
    x = x_ref[...]
    acc = jnp.full(x.shape, c_ref[0], jnp.float32)

    def step(i, acc):
        ci = c_ref[i]
        ci1 = c_ref[i + 1]
        mi = (ci + ci1) * 0.5
        return acc + jnp.where(x > mi, ci1 - ci, 0.0)

    o_ref[...] = lax.fori_loop(0, _N - 1, step, acc)


@functools.lru_cache(maxsize=None)
def _make_tc_quantize(rows: int):
    assert rows % _TC_BLK == 0
    return pl.pallas_call(
        _tc_body,
        grid=(rows // _TC_BLK,),
        in_specs=[
            pl.BlockSpec((_TC_BLK, 128), lambda i: (i, 0)),
            pl.BlockSpec(memory_space=pltpu.SMEM),
        ],
        out_specs=pl.BlockSpec((_TC_BLK, 128), lambda i: (i, 0)),
        out_shape=jax.ShapeDtypeStruct((rows, 128), jnp.float32),
    )


def kernel(sample, centers):
    x = sample.reshape(-1)
    c = centers.reshape(-1)
    batch = x.shape[0]
    n_tc = _TC_ROWS * 128
    n_sc = batch - n_tc
    out_sc = _make_sc_quantize(n_sc)(x[:n_sc], c)
    out_tc = _make_tc_quantize(_TC_ROWS)(x[n_sc:].reshape(_TC_ROWS, 128), c)
    out = jnp.concatenate([out_sc, out_tc.reshape(-1)])
    return out.reshape(-1, 1)


# speedup vs baseline: 1.7756x; 1.7756x over previous
import functools

import jax
import jax.numpy as jnp
from jax import lax
from jax.experimental import pallas as pl
from jax.experimental.pallas import tpu as pltpu
from jax.experimental.pallas import tpu_sc as plsc

_LANES = 16
_N = 256
_TABLE = 1024
_LO = -3.0
_WIDTH = 6.0 / _TABLE
_SCALE = _TABLE / 6.0


@functools.lru_cache(maxsize=None)
def _make_sc_quantize(batch: int):
    try:
        info = plsc.get_sparse_core_info()
        num_cores, num_subcores = info.num_cores, info.num_subcores
    except Exception:
        num_cores, num_subcores = 2, 16
    num_workers = num_cores * num_subcores
    assert batch % (num_workers * _LANES) == 0
    b_per_w = batch // num_workers
    half = b_per_w // 2
    widths = []
    w = _N // 2
    while w >= 1:
        widths.append(w)
        w //= 2

    mesh = plsc.VectorSubcoreMesh(
        core_axis_name="c",
        subcore_axis_name="s",
        num_cores=num_cores,
        num_subcores=num_subcores,
    )

    @functools.partial(
        pl.kernel,
        out_type=jax.ShapeDtypeStruct((batch,), jnp.float32),
        mesh=mesh,
        scratch_types=[
            pltpu.VMEM((b_per_w,), jnp.float32),
            pltpu.VMEM((b_per_w,), jnp.float32),
            pltpu.VMEM((_N + _LANES,), jnp.float32),
            pltpu.VMEM((_N,), jnp.float32),
            pltpu.VMEM((_TABLE,), jnp.int32),
            pltpu.VMEM((_TABLE,), jnp.float32),
            pltpu.SemaphoreType.DMA,
            pltpu.SemaphoreType.DMA,
            pltpu.SemaphoreType.DMA,
        ],
        compiler_params=pltpu.CompilerParams(needs_layout_passes=False),
    )
    def quantize(
        x_hbm, centers_hbm, out_hbm,
        x_v, o_v, cen_v, bnd_v, start_v, bval_v, sem_in, sem_out, sem_c,
    ):
        wid = lax.axis_index("s") * num_cores + lax.axis_index("c")
        base = wid * b_per_w
        in_copy = pltpu.async_copy(x_hbm.at[pl.ds(base, b_per_w)], x_v, sem_in)
        c_copy = pltpu.async_copy(centers_hbm, cen_v.at[pl.ds(0, _N)], sem_c)

        lane = lax.iota(jnp.int32, _LANES)
        inf = jnp.full((_LANES,), jnp.inf, jnp.float32)

        c_copy.wait()

        for j in range(_N // _LANES):
            lo = cen_v[pl.ds(j * _LANES, _LANES)]
            hi = plsc.load_gather(cen_v, [lane + (j * _LANES + 1)])
            mid = (lo + hi) * 0.5
            if j == _N // _LANES - 1:
                mid = jnp.where(lane == _LANES - 1, inf, mid)
            bnd_v[pl.ds(j * _LANES, _LANES)] = mid

        @plsc.parallel_loop(0, _TABLE // _LANES, 1, unroll=4)
        def _(j):
            g = (j * _LANES + lane).astype(jnp.float32) * _WIDTH + _LO
            pos = jnp.zeros((_LANES,), jnp.int32)
            for w in widths:
                mv = plsc.load_gather(bnd_v, [pos + (w - 1)])
                pos = jnp.where(mv < g, pos + w, pos)
            start_v[pl.ds(j * _LANES, _LANES)] = pos
            bval_v[pl.ds(j * _LANES, _LANES)] = plsc.load_gather(bnd_v, [pos])

        in_copy.wait()

        @plsc.parallel_loop(0, half // _LANES, 1, unroll=8)
        def _(i):
            x = x_v[pl.ds(i * _LANES, _LANES)]
            t = jnp.clip(((x - _LO) * _SCALE).astype(jnp.int32), 0, _TABLE - 1)
            s = plsc.load_gather(start_v, [t])
            bv = plsc.load_gather(bval_v, [t])
            pos = jnp.where(bv < x, s + 1, s)
            o_v[pl.ds(i * _LANES, _LANES)] = plsc.load_gather(cen_v, [pos])

        out_copy1 = pltpu.async_copy(
            o_v.at[pl.ds(0, half)], out_hbm.at[pl.ds(base, half)], sem_out
        )

        @plsc.parallel_loop(half // _LANES, b_per_w // _LANES, 1, unroll=8)
        def _(i):
            x = x_v[pl.ds(i * _LANES, _LANES)]
            t = jnp.clip(((x - _LO) * _SCALE).astype(jnp.int32), 0, _TABLE - 1)
            s = plsc.load_gather(start_v, [t])
            bv = plsc.load_gather(bval_v, [t])
            pos = jnp.where(bv < x, s + 1, s)
            o_v[pl.ds(i * _LANES, _LANES)] = plsc.load_gather(cen_v, [pos])

        out_copy2 = pltpu.async_copy(
            o_v.at[pl.ds(half, half)], out_hbm.at[pl.ds(base + half, half)], sem_out
        )
        out_copy1.wait()
        out_copy2.wait()

    return quantize


def kernel(sample, centers):
    x = sample.reshape(-1)
    c = centers.reshape(-1)
    out = _make_sc_quantize(x.shape[0])(x, c)
    return out.reshape(-1, 1)
